# baseline (speedup 1.0000x reference)
import functools

import numpy as np

import jax
import jax.numpy as jnp
from jax import lax
from jax.experimental import pallas as pl
from jax.experimental.pallas import tpu as pltpu
from jax.experimental.pallas import tpu_sc as plsc

N = 10000
D = 128
DH = D // 2
E = 320000
NC, NS, L = 2, 16, 16
NW = NC * NS
K = 128
EPAD = 327680
NCH_D = EPAD // NW // K
NCH_P = EPAD // NS // K
NPAD = 10240
RPT = NPAD // NS
RB = 1000


def _mesh():
    return plsc.VectorSubcoreMesh(
        core_axis_name="c", subcore_axis_name="s",
        num_cores=NC, num_subcores=NS)


def _deg(dstw):
    @functools.partial(
        pl.kernel,
        out_type=jax.ShapeDtypeStruct((NC, NPAD), jnp.float32),
        mesh=_mesh(),
        scratch_types=[
            pltpu.VMEM((NCH_D, K), jnp.int32),
            pltpu.VMEM((K,), jnp.float32),
            pltpu.VMEM((RPT,), jnp.float32),
            pltpu.VMEM_SHARED((NPAD,), jnp.float32),
        ],
        compiler_params=pltpu.CompilerParams(needs_layout_passes=False),
    )
    def body(dst_hbm, cnt_hbm, dst_v, ones_v, zero_v, cnt_sh):
        c = lax.axis_index("c")
        s = lax.axis_index("s")
        w = c * NS + s
        zero16 = jnp.zeros((L,), jnp.float32)
        one16 = jnp.ones((L,), jnp.float32)

        def fill(i, carry):
            ones_v[pl.ds(i * L, L)] = one16
            return carry
        lax.fori_loop(0, K // L, fill, 0)

        def zfill(i, carry):
            zero_v[pl.ds(i * L, L)] = zero16
            return carry
        lax.fori_loop(0, RPT // L, zfill, 0)

        base = s * RPT
        pltpu.sync_copy(zero_v, cnt_sh.at[pl.ds(base, RPT)])
        pltpu.sync_copy(dst_hbm.at[w], dst_v)
        plsc.subcore_barrier()

        def count_body(j, carry):
            pltpu.sync_copy(ones_v, cnt_sh.at[dst_v.at[j]], add=True)
            return carry
        lax.fori_loop(0, NCH_D, count_body, 0)

        plsc.subcore_barrier()
        pltpu.sync_copy(cnt_sh.at[pl.ds(base, RPT)],
                        cnt_hbm.at[c, pl.ds(base, RPT)])

    return body(dstw)


def _prop(xws, pkw):
    @functools.partial(
        pl.kernel,
        out_type=jax.ShapeDtypeStruct((NC, NPAD, D), jnp.float32),
        mesh=_mesh(),
        scratch_types=[
            pltpu.VMEM((NCH_D, K), jnp.int32),
            pltpu.VMEM((2, K), jnp.int32),
            pltpu.VMEM((2, K), jnp.int32),
            pltpu.VMEM((K, D), jnp.float32),
            pltpu.VMEM((K, D), jnp.float32),
            pltpu.VMEM_SHARED((NPAD, D), jnp.float32),
            pltpu.SemaphoreType.DMA,
            pltpu.SemaphoreType.DMA,
        ],
        compiler_params=pltpu.CompilerParams(needs_layout_passes=False),
    )
    def body(xws_hbm, pk_hbm, out_hbm,
             pk_v, src_row, dst_row, buf0, buf1, acc, gsem0, gsem1):
        c = lax.axis_index("c")
        s = lax.axis_index("s")
        w = c * NS + s
        zero16 = jnp.zeros((L,), jnp.float32)

        def zb(i, carry):
            r = i // (D // L)
            col = (i % (D // L)) * L
            buf0[r, pl.ds(col, L)] = zero16
            return carry
        lax.fori_loop(0, K * D // L, zb, 0)

        base = s * RPT
        for r in range(RPT // K):
            pltpu.sync_copy(buf0, acc.at[pl.ds(base + r * K, K)])

        pltpu.sync_copy(pk_hbm.at[w], pk_v)
        plsc.subcore_barrier()

        def unpack(j, slot):
            for g in range(K // L):
                v = pk_v[j, pl.ds(g * L, L)]
                src_row[slot, pl.ds(g * L, L)] = v & 16383
                dst_row[slot, pl.ds(g * L, L)] = lax.shift_right_logical(v, 14)

        def gather(slot, buf, sem):
            pltpu.async_copy(xws_hbm.at[src_row.at[slot]], buf, sem)

        def gwait(buf, sem):
            pltpu.make_async_copy(xws_hbm.at[src_row.at[0]], buf, sem).wait()

        def scatter(slot, buf):
            pltpu.sync_copy(buf, acc.at[dst_row.at[slot]], add=True)

        unpack(0, 0)
        gather(0, buf0, gsem0)
        unpack(1, 1)
        gather(1, buf1, gsem1)

        def pipe(g, carry):
            j = 2 * g
            gwait(buf0, gsem0)
            scatter(0, buf0)
            unpack(j + 2, 0)
            gather(0, buf0, gsem0)
            gwait(buf1, gsem1)
            scatter(1, buf1)
            unpack(j + 3, 1)
            gather(1, buf1, gsem1)
            return carry
        lax.fori_loop(0, NCH_D // 2 - 1, pipe, 0)

        gwait(buf0, gsem0)
        scatter(0, buf0)
        gwait(buf1, gsem1)
        scatter(1, buf1)

        plsc.subcore_barrier()
        pltpu.sync_copy(acc.at[pl.ds(base, RPT)],
                        out_hbm.at[c, pl.ds(base, RPT)])

    return body(xws, pkw)


def _dinv_of(cnt):
    deg = cnt[:, 0:1] + cnt[:, 1:2] + 1.0
    return lax.rsqrt(deg)


ER, EC = 80, EPAD // 80
ERB = ER // (N // RB)


def _k1a(x, W1, src2, dst2):
    def body(x_ref, w_ref, s_ref, d_ref, xw_ref, pk_ref):
        xw_ref[...] = jnp.dot(
            x_ref[...], w_ref[...], preferred_element_type=jnp.float32)
        pk_ref[...] = s_ref[...] | (d_ref[...] << 14)

    return pl.pallas_call(
        body,
        grid=(N // RB,),
        in_specs=[
            pl.BlockSpec((RB, D), lambda i: (i, 0)),
            pl.BlockSpec((D, D), lambda i: (0, 0)),
            pl.BlockSpec((ERB, EC), lambda i: (i, 0)),
            pl.BlockSpec((ERB, EC), lambda i: (i, 0)),
        ],
        out_specs=[
            pl.BlockSpec((RB, D), lambda i: (i, 0)),
            pl.BlockSpec((ERB, EC), lambda i: (i, 0)),
        ],
        out_shape=[
            jax.ShapeDtypeStruct((N, D), jnp.float32),
            jax.ShapeDtypeStruct((ER, EC), jnp.int32),
        ],
    )(x, W1, src2, dst2)


def _k1b(cnt_t, xw1):
    def body(cnt_ref, xw_ref, xws_ref):
        xws_ref[...] = _dinv_of(cnt_ref[...]) * xw_ref[...]

    return pl.pallas_call(
        body,
        grid=(N // RB,),
        in_specs=[
            pl.BlockSpec((RB, NC), lambda i: (i, 0)),
            pl.BlockSpec((RB, D), lambda i: (i, 0)),
        ],
        out_specs=pl.BlockSpec((RB, D), lambda i: (i, 0)),
        out_shape=jax.ShapeDtypeStruct((N, D), jnp.float32),
    )(cnt_t, xw1)


def _k2(parts, xws1, cnt_t, b1, W2):
    def body(p_ref, xws_ref, cnt_ref, b_ref, w_ref, o_ref):
        dinv = _dinv_of(cnt_ref[...])
        ssum = p_ref[0] + p_ref[1] + xws_ref[...]
        h = jnp.maximum(dinv * ssum + b_ref[...], 0.0)
        o_ref[...] = dinv * jnp.dot(
            h, w_ref[...], preferred_element_type=jnp.float32)

    return pl.pallas_call(
        body,
        grid=(N // RB,),
        in_specs=[
            pl.BlockSpec((NC, RB, D), lambda i: (0, i, 0)),
            pl.BlockSpec((RB, D), lambda i: (i, 0)),
            pl.BlockSpec((RB, NC), lambda i: (i, 0)),
            pl.BlockSpec((1, D), lambda i: (0, 0)),
            pl.BlockSpec((D, D), lambda i: (0, 0)),
        ],
        out_specs=pl.BlockSpec((RB, D), lambda i: (i, 0)),
        out_shape=jax.ShapeDtypeStruct((N, D), jnp.float32),
    )(parts, xws1, cnt_t, b1, W2)


def _k3(parts, xws2, cnt_t, b2):
    def body(p_ref, xws_ref, cnt_ref, b_ref, o_ref):
        ssum = p_ref[0] + p_ref[1] + xws_ref[...]
        o_ref[...] = _dinv_of(cnt_ref[...]) * ssum + b_ref[...]

    return pl.pallas_call(
        body,
        grid=(N // RB,),
        in_specs=[
            pl.BlockSpec((NC, RB, D), lambda i: (0, i, 0)),
            pl.BlockSpec((RB, D), lambda i: (i, 0)),
            pl.BlockSpec((RB, NC), lambda i: (i, 0)),
            pl.BlockSpec((1, D), lambda i: (0, 0)),
        ],
        out_specs=pl.BlockSpec((RB, D), lambda i: (i, 0)),
        out_shape=jax.ShapeDtypeStruct((N, D), jnp.float32),
    )(parts, xws2, cnt_t, b2)


def kernel(x, edge_index, W1, b1, W2, b2):
    src = edge_index[0].astype(jnp.int32)
    dst = edge_index[1].astype(jnp.int32)
    pad = EPAD - E
    padi = np.arange(pad, dtype=np.int32)
    srcf = jnp.concatenate([src, jnp.asarray(padi % N)])
    dstf = jnp.concatenate([dst, jnp.asarray(N + padi % (NPAD - N))])

    cnt = _deg(dstf.reshape(NW, NCH_D, K))
    xw1, pk2 = _k1a(x, W1, srcf.reshape(ER, EC), dstf.reshape(ER, EC))
    pkw = pk2.reshape(NW, NCH_D, K)
    cnt_t = cnt.T
    xws1 = _k1b(cnt_t, xw1)
    parts1 = _prop(xws1, pkw)
    xws2 = _k2(parts1, xws1, cnt_t, b1.reshape(1, D), W2)
    parts2 = _prop(xws2, pkw)
    return _k3(parts2, xws2, cnt_t, b2.reshape(1, D))


# speedup vs baseline: 1.0101x; 1.0101x over previous
import functools

import jax
import jax.numpy as jnp
from jax import lax
from jax.experimental import pallas as pl
from jax.experimental.pallas import tpu as pltpu
from jax.experimental.pallas import tpu_sc as plsc

N = 10000
D = 128
E = 320000
NC, NS, L = 2, 16, 16
NW = NC * NS
K = 128
EPAD = 327680
NCH = EPAD // NW // K
NPAD = 10240
RPT = NPAD // NS
RB = 1000


def _mesh():
    return plsc.VectorSubcoreMesh(
        core_axis_name="c", subcore_axis_name="s",
        num_cores=NC, num_subcores=NS)


def _deg(pkw):
    @functools.partial(
        pl.kernel,
        out_type=jax.ShapeDtypeStruct((NC, NPAD), jnp.float32),
        mesh=_mesh(),
        scratch_types=[
            pltpu.VMEM((NCH, K), jnp.int32),
            pltpu.VMEM((NCH, K), jnp.int32),
            pltpu.VMEM((K,), jnp.float32),
            pltpu.VMEM((RPT,), jnp.float32),
            pltpu.VMEM_SHARED((NPAD,), jnp.float32),
        ],
        compiler_params=pltpu.CompilerParams(needs_layout_passes=False),
    )
    def body(pk_hbm, cnt_hbm, pk_v, dst_v, ones_v, zero_v, cnt_sh):
        c = lax.axis_index("c")
        s = lax.axis_index("s")
        w = c * NS + s
        zero16 = jnp.zeros((L,), jnp.float32)
        one16 = jnp.ones((L,), jnp.float32)

        def fill(i, carry):
            ones_v[pl.ds(i * L, L)] = one16
            return carry
        lax.fori_loop(0, K // L, fill, 0)

        def zfill(i, carry):
            zero_v[pl.ds(i * L, L)] = zero16
            return carry
        lax.fori_loop(0, RPT // L, zfill, 0)

        base = s * RPT
        pltpu.sync_copy(zero_v, cnt_sh.at[pl.ds(base, RPT)])
        pltpu.sync_copy(pk_hbm.at[w], pk_v)

        def unpack(i, carry):
            r = i // (K // L)
            col = (i % (K // L)) * L
            v = pk_v[r, pl.ds(col, L)]
            dst_v[r, pl.ds(col, L)] = lax.shift_right_logical(v, 14)
            return carry
        lax.fori_loop(0, NCH * K // L, unpack, 0)
        plsc.subcore_barrier()

        def count_body(j, carry):
            pltpu.sync_copy(ones_v, cnt_sh.at[dst_v.at[j]], add=True)
            return carry
        lax.fori_loop(0, NCH, count_body, 0)

        plsc.subcore_barrier()
        pltpu.sync_copy(cnt_sh.at[pl.ds(base, RPT)],
                        cnt_hbm.at[c, pl.ds(base, RPT)])

    return body(pkw)


def _prop(xws, pkw):
    @functools.partial(
        pl.kernel,
        out_type=jax.ShapeDtypeStruct((NC, NPAD, D), jnp.float32),
        mesh=_mesh(),
        scratch_types=[
            pltpu.VMEM((NCH, K), jnp.int32),
            pltpu.VMEM((2, K), jnp.int32),
            pltpu.VMEM((2, K), jnp.int32),
            pltpu.VMEM((K, D), jnp.float32),
            pltpu.VMEM((K, D), jnp.float32),
            pltpu.VMEM_SHARED((NPAD, D), jnp.float32),
            pltpu.SemaphoreType.DMA,
            pltpu.SemaphoreType.DMA,
        ],
        compiler_params=pltpu.CompilerParams(needs_layout_passes=False),
    )
    def body(xws_hbm, pk_hbm, out_hbm,
             pk_v, src_row, dst_row, buf0, buf1, acc, gsem0, gsem1):
        c = lax.axis_index("c")
        s = lax.axis_index("s")
        w = c * NS + s
        zero16 = jnp.zeros((L,), jnp.float32)

        def zb(i, carry):
            r = i // (D // L)
            col = (i % (D // L)) * L
            buf0[r, pl.ds(col, L)] = zero16
            return carry
        lax.fori_loop(0, K * D // L, zb, 0)

        base = s * RPT
        for r in range(RPT // K):
            pltpu.sync_copy(buf0, acc.at[pl.ds(base + r * K, K)])

        pltpu.sync_copy(pk_hbm.at[w], pk_v)
        plsc.subcore_barrier()

        def unpack(j, slot):
            for g in range(K // L):
                v = pk_v[j, pl.ds(g * L, L)]
                src_row[slot, pl.ds(g * L, L)] = v & 16383
                dst_row[slot, pl.ds(g * L, L)] = lax.shift_right_logical(v, 14)

        def gather(slot, buf, sem):
            pltpu.async_copy(xws_hbm.at[src_row.at[slot]], buf, sem)

        def gwait(buf, sem):
            pltpu.make_async_copy(xws_hbm.at[src_row.at[0]], buf, sem).wait()

        def scatter(slot, buf):
            pltpu.sync_copy(buf, acc.at[dst_row.at[slot]], add=True)

        unpack(0, 0)
        gather(0, buf0, gsem0)
        unpack(1, 1)
        gather(1, buf1, gsem1)

        def pipe(g, carry):
            j = 2 * g
            gwait(buf0, gsem0)
            scatter(0, buf0)
            unpack(j + 2, 0)
            gather(0, buf0, gsem0)
            gwait(buf1, gsem1)
            scatter(1, buf1)
            unpack(j + 3, 1)
            gather(1, buf1, gsem1)
            return carry
        lax.fori_loop(0, NCH // 2 - 1, pipe, 0)

        gwait(buf0, gsem0)
        scatter(0, buf0)
        gwait(buf1, gsem1)
        scatter(1, buf1)

        plsc.subcore_barrier()
        pltpu.sync_copy(acc.at[pl.ds(base, RPT)],
                        out_hbm.at[c, pl.ds(base, RPT)])

    return body(xws, pkw)


def _k1(cnt_t, x, W1):
    def body(cnt_ref, x_ref, w_ref, dinv_ref, xws_ref):
        cnt = cnt_ref[...]
        deg = cnt[:, 0:1] + cnt[:, 1:2] + 1.0
        dinv = lax.rsqrt(deg)
        dinv_ref[...] = jnp.broadcast_to(dinv, (RB, D))
        xws_ref[...] = dinv * jnp.dot(
            x_ref[...], w_ref[...], preferred_element_type=jnp.float32)

    return pl.pallas_call(
        body,
        grid=(N // RB,),
        in_specs=[
            pl.BlockSpec((RB, NC), lambda i: (i, 0)),
            pl.BlockSpec((RB, D), lambda i: (i, 0)),
            pl.BlockSpec((D, D), lambda i: (0, 0)),
        ],
        out_specs=[
            pl.BlockSpec((RB, D), lambda i: (i, 0)),
            pl.BlockSpec((RB, D), lambda i: (i, 0)),
        ],
        out_shape=[
            jax.ShapeDtypeStruct((N, D), jnp.float32),
            jax.ShapeDtypeStruct((N, D), jnp.float32),
        ],
    )(cnt_t, x, W1)


def _k2(parts, xws1, dinv_bc, b1, W2):
    def body(p_ref, xws_ref, dinv_ref, b_ref, w_ref, o_ref):
        ssum = p_ref[0] + p_ref[1] + xws_ref[...]
        h = jnp.maximum(dinv_ref[...] * ssum + b_ref[...], 0.0)
        o_ref[...] = dinv_ref[...] * jnp.dot(
            h, w_ref[...], preferred_element_type=jnp.float32)

    return pl.pallas_call(
        body,
        grid=(N // RB,),
        in_specs=[
            pl.BlockSpec((NC, RB, D), lambda i: (0, i, 0)),
            pl.BlockSpec((RB, D), lambda i: (i, 0)),
            pl.BlockSpec((RB, D), lambda i: (i, 0)),
            pl.BlockSpec((1, D), lambda i: (0, 0)),
            pl.BlockSpec((D, D), lambda i: (0, 0)),
        ],
        out_specs=pl.BlockSpec((RB, D), lambda i: (i, 0)),
        out_shape=jax.ShapeDtypeStruct((N, D), jnp.float32),
    )(parts, xws1, dinv_bc, b1, W2)


def _k3(parts, xws2, dinv_bc, b2):
    def body(p_ref, xws_ref, dinv_ref, b_ref, o_ref):
        ssum = p_ref[0] + p_ref[1] + xws_ref[...]
        o_ref[...] = dinv_ref[...] * ssum + b_ref[...]

    return pl.pallas_call(
        body,
        grid=(N // RB,),
        in_specs=[
            pl.BlockSpec((NC, RB, D), lambda i: (0, i, 0)),
            pl.BlockSpec((RB, D), lambda i: (i, 0)),
            pl.BlockSpec((RB, D), lambda i: (i, 0)),
            pl.BlockSpec((1, D), lambda i: (0, 0)),
        ],
        out_specs=pl.BlockSpec((RB, D), lambda i: (i, 0)),
        out_shape=jax.ShapeDtypeStruct((N, D), jnp.float32),
    )(parts, xws2, dinv_bc, b2)


def kernel(x, edge_index, W1, b1, W2, b2):
    src = edge_index[0].astype(jnp.int32)
    dst = edge_index[1].astype(jnp.int32)
    pad = EPAD - E
    pk = src | (dst << 14)
    padi = jnp.arange(pad, dtype=jnp.int32)
    pad_pk = (padi % N) | ((N + padi % (NPAD - N)) << 14)
    pkw = jnp.concatenate([pk, pad_pk]).reshape(NW, NCH, K)

    cnt = _deg(pkw)
    dinv_bc, xws1 = _k1(cnt.T, x, W1)
    parts1 = _prop(xws1, pkw)
    xws2 = _k2(parts1, xws1, dinv_bc, b1.reshape(1, D), W2)
    parts2 = _prop(xws2, pkw)
    return _k3(parts2, xws2, dinv_bc, b2.reshape(1, D))
